# baseline (speedup 1.0000x reference)
import functools

import jax
import jax.numpy as jnp
from jax import lax
from jax.experimental import pallas as pl
from jax.experimental.pallas import tpu as pltpu
from jax.experimental.pallas import tpu_sc as plsc

_DIM = 2048
_E = 64
_K = 8
_G = 8
_KG = 4
_GS = _E // _G
_NC = 2
_NS = 16
_NW = _NC * _NS
_L = 16
_NEG = float("-inf")


def _tc_route(s, w_out_ref, i_out_ref, lo):
    bt = s.shape[1]
    neg = jnp.float32(-jnp.inf)

    gms = [jnp.max(s[g * _GS:(g + 1) * _GS], axis=0, keepdims=True)
           for g in range(_G)]

    picked = [jnp.zeros((1, bt), jnp.bool_) for _ in range(_G)]
    for _ in range(_KG):
        mm = jnp.where(picked[0], neg, gms[0])
        for g in range(1, _G):
            mm = jnp.maximum(mm, jnp.where(picked[g], neg, gms[g]))
        gsel = jnp.full((1, bt), jnp.float32(_G))
        for g in reversed(range(_G)):
            gsel = jnp.where((~picked[g]) & (gms[g] == mm),
                             jnp.float32(g), gsel)
        for g in range(_G):
            picked[g] = picked[g] | (gsel == jnp.float32(g))

    sm = jnp.concatenate(
        [jnp.where(picked[g], s[g * _GS:(g + 1) * _GS], jnp.float32(0.0))
         for g in range(_G)], axis=0)

    rowf = jax.lax.broadcasted_iota(jnp.int32, (_E, bt), 0).astype(jnp.float32)
    wvals = []
    widxs = []
    for _ in range(_K):
        m = jnp.max(sm, axis=0, keepdims=True)
        idx = jnp.min(jnp.where(sm == m, rowf, jnp.float32(_E)),
                      axis=0, keepdims=True)
        hit = rowf == idx
        sm = jnp.where(hit, jnp.float32(-1.0), sm)
        wvals.append(m)
        widxs.append(idx)

    wts = jnp.concatenate(wvals, axis=0)
    idxs = jnp.concatenate(widxs, axis=0)
    wts = wts / jnp.sum(wts, axis=0, keepdims=True)
    w_out_ref[:, lo:lo + bt] = wts
    i_out_ref[:, lo:lo + bt] = idxs.astype(jnp.int32)


def _gate_block(x_ref, w_ref, w_out_ref, i_out_ref):
    w = w_ref[...]
    bt = x_ref.shape[0]
    sub = 1024
    scores = []
    for j in range(bt // sub):
        x = x_ref[j * sub:(j + 1) * sub, :]
        logits = jax.lax.dot_general(
            w, x, (((1,), (1,)), ((), ())),
            preferred_element_type=jnp.float32)
        scores.append(jax.nn.sigmoid(logits))
    for j, s in enumerate(scores):
        _tc_route(s, w_out_ref, i_out_ref, j * sub)


def _score_block(x_ref, w_ref, s_ref):
    x = x_ref[...]
    w = w_ref[...]
    logits = jax.lax.dot_general(
        w, x, (((1,), (1,)), ((), ())), preferred_element_type=jnp.float32)
    s_ref[0, :, :] = jax.nn.sigmoid(logits)


def _merge(a, b):
    c = a[0] >= b[0]
    return jnp.where(c, a[0], b[0]), jnp.where(c, a[1], b[1])


def _tree_max(pairs):
    cur = list(pairs)
    while len(cur) > 1:
        nxt = [_merge(cur[2 * j], cur[2 * j + 1])
               for j in range(len(cur) // 2)]
        if len(cur) % 2:
            nxt.append(cur[-1])
        cur = nxt
    return cur[0]


def _sc_route_body(scores_hbm, w_out_hbm, i_out_hbm, svmem, smv, wv_v, iv_v):
    cpw = svmem.shape[1]
    wid = lax.axis_index("s") * _NC + lax.axis_index("c")
    pltpu.sync_copy(scores_hbm.at[wid], svmem)
    lanes = lax.iota(jnp.int32, _L)

    def batch(i, carry):
        base = i * _L
        s = [svmem[e, pl.ds(base, _L)] for e in range(_E)]

        gm = []
        for g in range(_G):
            t = s[_GS * g]
            for j in range(1, _GS):
                t = jnp.maximum(t, s[_GS * g + j])
            gm.append(t)

        picked = [jnp.zeros((_L,), jnp.bool_) for _ in range(_G)]
        for _ in range(_KG):
            leaves = [(jnp.where(picked[g], _NEG, gm[g]),
                       jnp.full((_L,), jnp.float32(g))) for g in range(_G)]
            _, gsel = _tree_max(leaves)
            for g in range(_G):
                picked[g] = picked[g] | (gsel == jnp.float32(g))

        for e in range(_E):
            smv[pl.ds(e * _L, _L)] = jnp.where(
                picked[e // _GS], s[e], jnp.float32(0.0))

        wvals = []
        ivals = []
        for _ in range(_K):
            subs = []
            for g in range(_G):
                leaves = [(smv[pl.ds((_GS * g + j) * _L, _L)],
                           jnp.full((_L,), jnp.float32(_GS * g + j)))
                          for j in range(_GS)]
                subs.append(_tree_max(leaves))
            m, midx = _tree_max(subs)
            midx_i = midx.astype(jnp.int32)
            plsc.store_scatter(smv, [midx_i * _L + lanes],
                               jnp.full((_L,), jnp.float32(-1.0)))
            wvals.append(m)
            ivals.append(midx_i)

        tot = wvals[0]
        for k in range(1, _K):
            tot = tot + wvals[k]
        inv = jnp.float32(1.0) / tot
        for k in range(_K):
            wv_v[k, pl.ds(base, _L)] = wvals[k] * inv
            iv_v[k, pl.ds(base, _L)] = ivals[k]
        return carry

    lax.fori_loop(0, cpw // _L, batch, jnp.int32(0))
    pltpu.sync_copy(wv_v, w_out_hbm.at[wid])
    pltpu.sync_copy(iv_v, i_out_hbm.at[wid])


@jax.jit
def kernel(x, weight):
    t = x.shape[0]
    scpart = t // 4
    tcpart = t - scpart
    fbt = 2048
    sbt = scpart // _NW

    w1, i1 = pl.pallas_call(
        _gate_block,
        grid=(tcpart // fbt,),
        in_specs=[
            pl.BlockSpec((fbt, _DIM), lambda i: (i, 0)),
            pl.BlockSpec((_E, _DIM), lambda i: (0, 0)),
        ],
        out_specs=[
            pl.BlockSpec((_K, fbt), lambda i: (0, i)),
            pl.BlockSpec((_K, fbt), lambda i: (0, i)),
        ],
        out_shape=[
            jax.ShapeDtypeStruct((_K, tcpart), jnp.float32),
            jax.ShapeDtypeStruct((_K, tcpart), jnp.int32),
        ],
        compiler_params=pltpu.CompilerParams(
            dimension_semantics=("parallel",)),
    )(x, weight)

    noff_blocks = tcpart // sbt
    scores = pl.pallas_call(
        _score_block,
        grid=(_NW,),
        in_specs=[
            pl.BlockSpec((sbt, _DIM), lambda i: (i + noff_blocks, 0)),
            pl.BlockSpec((_E, _DIM), lambda i: (0, 0)),
        ],
        out_specs=pl.BlockSpec((1, _E, sbt), lambda i: (i, 0, 0)),
        out_shape=jax.ShapeDtypeStruct((_NW, _E, sbt), jnp.float32),
        compiler_params=pltpu.CompilerParams(
            dimension_semantics=("parallel",)),
    )(x, weight)

    mesh = plsc.VectorSubcoreMesh(core_axis_name="c", subcore_axis_name="s")
    sc_route = functools.partial(
        pl.kernel,
        mesh=mesh,
        out_type=[
            jax.ShapeDtypeStruct((_NW, _K, sbt), jnp.float32),
            jax.ShapeDtypeStruct((_NW, _K, sbt), jnp.int32),
        ],
        scratch_types=[
            pltpu.VMEM((_E, sbt), jnp.float32),
            pltpu.VMEM((_E * _L,), jnp.float32),
            pltpu.VMEM((_K, sbt), jnp.float32),
            pltpu.VMEM((_K, sbt), jnp.int32),
        ],
        compiler_params=pltpu.CompilerParams(
            needs_layout_passes=False, skip_device_barrier=True),
    )(_sc_route_body)
    w3, i3 = sc_route(scores)

    w = jnp.concatenate(
        [w1.T, w3.transpose(0, 2, 1).reshape(scpart, _K)], axis=0)
    i = jnp.concatenate(
        [i1.T, i3.transpose(0, 2, 1).reshape(scpart, _K)], axis=0)
    return w, i


# speedup vs baseline: 1.0253x; 1.0253x over previous
import functools

import jax
import jax.numpy as jnp
from jax import lax
from jax.experimental import pallas as pl
from jax.experimental.pallas import tpu as pltpu
from jax.experimental.pallas import tpu_sc as plsc

_DIM = 2048
_E = 64
_K = 8
_G = 8
_KG = 4
_GS = _E // _G
_NC = 2
_NS = 16
_NW = _NC * _NS
_L = 16
_NEG = float("-inf")


def _tc_route(s, w_out_ref, i_out_ref, lo):
    bt = s.shape[1]
    neg = jnp.float32(-jnp.inf)

    gms = [jnp.max(s[g * _GS:(g + 1) * _GS], axis=0, keepdims=True)
           for g in range(_G)]

    picked = [jnp.zeros((1, bt), jnp.bool_) for _ in range(_G)]
    for _ in range(_KG):
        mm = jnp.where(picked[0], neg, gms[0])
        for g in range(1, _G):
            mm = jnp.maximum(mm, jnp.where(picked[g], neg, gms[g]))
        gsel = jnp.full((1, bt), jnp.float32(_G))
        for g in reversed(range(_G)):
            gsel = jnp.where((~picked[g]) & (gms[g] == mm),
                             jnp.float32(g), gsel)
        for g in range(_G):
            picked[g] = picked[g] | (gsel == jnp.float32(g))

    sm = jnp.concatenate(
        [jnp.where(picked[g], s[g * _GS:(g + 1) * _GS], jnp.float32(0.0))
         for g in range(_G)], axis=0)

    rowf = jax.lax.broadcasted_iota(jnp.int32, (_E, bt), 0).astype(jnp.float32)
    wvals = []
    widxs = []
    for _ in range(_K):
        m = jnp.max(sm, axis=0, keepdims=True)
        idx = jnp.min(jnp.where(sm == m, rowf, jnp.float32(_E)),
                      axis=0, keepdims=True)
        hit = rowf == idx
        sm = jnp.where(hit, jnp.float32(-1.0), sm)
        wvals.append(m)
        widxs.append(idx)

    wts = jnp.concatenate(wvals, axis=0)
    idxs = jnp.concatenate(widxs, axis=0)
    wts = wts / jnp.sum(wts, axis=0, keepdims=True)
    w_out_ref[:, lo:lo + bt] = wts
    i_out_ref[:, lo:lo + bt] = idxs.astype(jnp.int32)


def _gate_block(x_ref, w_ref, w_out_ref, i_out_ref):
    w = w_ref[...]
    bt = x_ref.shape[0]
    sub = 1024
    scores = []
    for j in range(bt // sub):
        x = x_ref[j * sub:(j + 1) * sub, :]
        logits = jax.lax.dot_general(
            w, x, (((1,), (1,)), ((), ())),
            preferred_element_type=jnp.float32)
        scores.append(jax.nn.sigmoid(logits))
    for j, s in enumerate(scores):
        _tc_route(s, w_out_ref, i_out_ref, j * sub)


def _score_block(x_ref, w_ref, s_ref):
    x = x_ref[...]
    w = w_ref[...]
    logits = jax.lax.dot_general(
        w, x, (((1,), (1,)), ((), ())), preferred_element_type=jnp.float32)
    s_ref[0, :, :] = jax.nn.sigmoid(logits)


def _merge(a, b):
    c = a[0] >= b[0]
    return jnp.where(c, a[0], b[0]), jnp.where(c, a[1], b[1])


def _tree_max(pairs):
    cur = list(pairs)
    while len(cur) > 1:
        nxt = [_merge(cur[2 * j], cur[2 * j + 1])
               for j in range(len(cur) // 2)]
        if len(cur) % 2:
            nxt.append(cur[-1])
        cur = nxt
    return cur[0]


def _sc_route_body(scores_hbm, w_out_hbm, i_out_hbm, svmem, smv, wv_v, iv_v):
    cpw = svmem.shape[1]
    wid = lax.axis_index("s") * _NC + lax.axis_index("c")
    pltpu.sync_copy(scores_hbm.at[wid], svmem)
    lanes = lax.iota(jnp.int32, _L)

    def batch(i, carry):
        base = i * _L
        s = [svmem[e, pl.ds(base, _L)] for e in range(_E)]

        gm = []
        for g in range(_G):
            t = s[_GS * g]
            for j in range(1, _GS):
                t = jnp.maximum(t, s[_GS * g + j])
            gm.append(t)

        picked = [jnp.zeros((_L,), jnp.bool_) for _ in range(_G)]
        for _ in range(_KG):
            leaves = [(jnp.where(picked[g], _NEG, gm[g]),
                       jnp.full((_L,), jnp.float32(g))) for g in range(_G)]
            _, gsel = _tree_max(leaves)
            for g in range(_G):
                picked[g] = picked[g] | (gsel == jnp.float32(g))

        for e in range(_E):
            smv[pl.ds(e * _L, _L)] = jnp.where(
                picked[e // _GS], s[e], jnp.float32(0.0))

        wvals = []
        ivals = []
        for _ in range(_K):
            subs = []
            for g in range(_G):
                leaves = [(smv[pl.ds((_GS * g + j) * _L, _L)],
                           jnp.full((_L,), jnp.float32(_GS * g + j)))
                          for j in range(_GS)]
                subs.append(_tree_max(leaves))
            m, midx = _tree_max(subs)
            midx_i = midx.astype(jnp.int32)
            plsc.store_scatter(smv, [midx_i * _L + lanes],
                               jnp.full((_L,), jnp.float32(-1.0)))
            wvals.append(m)
            ivals.append(midx_i)

        tot = wvals[0]
        for k in range(1, _K):
            tot = tot + wvals[k]
        inv = jnp.float32(1.0) / tot
        for k in range(_K):
            wv_v[k, pl.ds(base, _L)] = wvals[k] * inv
            iv_v[k, pl.ds(base, _L)] = ivals[k]
        return carry

    lax.fori_loop(0, cpw // _L, batch, jnp.int32(0))
    pltpu.sync_copy(wv_v, w_out_hbm.at[wid])
    pltpu.sync_copy(iv_v, i_out_hbm.at[wid])


@jax.jit
def kernel(x, weight):
    t = x.shape[0]
    scpart = t // 2
    tcpart = t - scpart
    fbt = 2048
    sbt = scpart // _NW

    w1, i1 = pl.pallas_call(
        _gate_block,
        grid=(tcpart // fbt,),
        in_specs=[
            pl.BlockSpec((fbt, _DIM), lambda i: (i, 0)),
            pl.BlockSpec((_E, _DIM), lambda i: (0, 0)),
        ],
        out_specs=[
            pl.BlockSpec((_K, fbt), lambda i: (0, i)),
            pl.BlockSpec((_K, fbt), lambda i: (0, i)),
        ],
        out_shape=[
            jax.ShapeDtypeStruct((_K, tcpart), jnp.float32),
            jax.ShapeDtypeStruct((_K, tcpart), jnp.int32),
        ],
        compiler_params=pltpu.CompilerParams(
            dimension_semantics=("parallel",)),
    )(x, weight)

    noff_blocks = tcpart // sbt
    scores = pl.pallas_call(
        _score_block,
        grid=(_NW,),
        in_specs=[
            pl.BlockSpec((sbt, _DIM), lambda i: (i + noff_blocks, 0)),
            pl.BlockSpec((_E, _DIM), lambda i: (0, 0)),
        ],
        out_specs=pl.BlockSpec((1, _E, sbt), lambda i: (i, 0, 0)),
        out_shape=jax.ShapeDtypeStruct((_NW, _E, sbt), jnp.float32),
        compiler_params=pltpu.CompilerParams(
            dimension_semantics=("parallel",)),
    )(x, weight)

    mesh = plsc.VectorSubcoreMesh(core_axis_name="c", subcore_axis_name="s")
    sc_route = functools.partial(
        pl.kernel,
        mesh=mesh,
        out_type=[
            jax.ShapeDtypeStruct((_NW, _K, sbt), jnp.float32),
            jax.ShapeDtypeStruct((_NW, _K, sbt), jnp.int32),
        ],
        scratch_types=[
            pltpu.VMEM((_E, sbt), jnp.float32),
            pltpu.VMEM((_E * _L,), jnp.float32),
            pltpu.VMEM((_K, sbt), jnp.float32),
            pltpu.VMEM((_K, sbt), jnp.int32),
        ],
        compiler_params=pltpu.CompilerParams(
            needs_layout_passes=False, skip_device_barrier=True),
    )(_sc_route_body)
    w3, i3 = sc_route(scores)

    w = jnp.concatenate(
        [w1.T, w3.transpose(0, 2, 1).reshape(scpart, _K)], axis=0)
    i = jnp.concatenate(
        [i1.T, i3.transpose(0, 2, 1).reshape(scpart, _K)], axis=0)
    return w, i
